# baseline (speedup 1.0000x reference)
import dataclasses
import functools

import numpy as np
import jax
import jax.numpy as jnp
from jax import lax
from jax.experimental import pallas as pl
from jax.experimental.pallas import tpu as pltpu
from jax.experimental.pallas import tpu_sc as plsc

_SHUFFLE_CHANNEL = 2048
_TOTAL = 8192
_NC, _NS, _L = 2, 16, 16
_NW = _NC * _NS
_ROWS = 4 * 1024
_RPW = _ROWS // _NW
_NBUF = 8
_PREF = 5


def _build_index() -> np.ndarray:
    pkey = jax.random.key(42)
    random_sort = jax.random.permutation(pkey, _TOTAL)[:_SHUFFLE_CHANNEL]
    random_index = jnp.sort(random_sort)
    rs = np.asarray(random_sort).astype(np.int32)
    ri = np.asarray(random_index).astype(np.int32)
    return np.concatenate([rs, ri])


_IDX = _build_index()


@jax.jit
def _sc_shuffle(x2d, idx):
    mesh = plsc.VectorSubcoreMesh(
        core_axis_name="c", subcore_axis_name="s",
        num_cores=_NC, num_subcores=_NS,
    )

    cp = pltpu.CompilerParams()
    if "needs_layout_passes" in pltpu.CompilerParams.__dataclass_fields__:
        cp = dataclasses.replace(cp, needs_layout_passes=False)

    @functools.partial(
        pl.kernel,
        mesh=mesh,
        compiler_params=cp,
        out_type=jax.ShapeDtypeStruct((_ROWS, _TOTAL), jnp.float32),
        scratch_types=(
            [pltpu.VMEM((2 * _SHUFFLE_CHANNEL,), jnp.int32)]
            + [pltpu.VMEM((_TOTAL,), jnp.float32) for _ in range(_NBUF)]
            + [pltpu.VMEM((_SHUFFLE_CHANNEL,), jnp.float32)]
            + [pltpu.SemaphoreType.DMA for _ in range(2 * _NBUF)]
        ),
    )
    def k(x_hbm, idx_hbm, o_hbm, idx_v, *rest):
        bufs = rest[:_NBUF]
        g_v = rest[_NBUF]
        sins = rest[_NBUF + 1:2 * _NBUF + 1]
        souts = rest[2 * _NBUF + 1:]
        wid = lax.axis_index("s") * _NC + lax.axis_index("c")
        base = wid * _RPW
        pltpu.sync_copy(idx_hbm, idx_v)

        def fix(buf):
            @plsc.parallel_loop(0, _SHUFFLE_CHANNEL, step=_L, unroll=16)
            def _gather(j):
                g_v[pl.ds(j, _L)] = plsc.load_gather(buf, [idx_v[pl.ds(j, _L)]])

            @plsc.parallel_loop(0, _SHUFFLE_CHANNEL, step=_L, unroll=16)
            def _scatter(j):
                plsc.store_scatter(buf, [idx_v[pl.ds(_SHUFFLE_CHANNEL + j, _L)]],
                                   g_v[pl.ds(j, _L)])

        for p in range(_PREF):
            pltpu.async_copy(x_hbm.at[base + p], bufs[p], sins[p])

        @pl.loop(0, _RPW, step=_NBUF)
        def _grp(g):
            for kk in range(_NBUF):
                b = g + kk
                kn = (kk + _PREF) % _NBUF

                pltpu.make_async_copy(x_hbm.at[base + b], bufs[kk], sins[kk]).wait()
                fix(bufs[kk])
                pltpu.async_copy(bufs[kk], o_hbm.at[base + b], souts[kk])

                @pl.when(b >= _NBUF - _PREF)
                def _retire():
                    pltpu.make_async_copy(
                        bufs[kn], o_hbm.at[base + b - (_NBUF - _PREF)],
                        souts[kn]).wait()

                @pl.when(b + _PREF < _RPW)
                def _prefetch():
                    pltpu.async_copy(x_hbm.at[base + b + _PREF], bufs[kn],
                                     sins[kn])

        for bb in range(_RPW - (_NBUF - _PREF), _RPW):
            kk = bb % _NBUF
            pltpu.make_async_copy(bufs[kk], o_hbm.at[base + bb],
                                  souts[kk]).wait()

    return k(x2d, idx)


def kernel(input):
    x2d = input.reshape(_ROWS, _TOTAL)
    out = _sc_shuffle(x2d, jnp.asarray(_IDX))
    return out.reshape(input.shape)
